# baseline (speedup 1.0000x reference)
import functools

import jax
import jax.numpy as jnp
from jax import lax
from jax.experimental import pallas as pl
from jax.experimental.pallas import tpu as pltpu
from jax.experimental.pallas import tpu_sc as plsc

BATCH = 4096
HIST = 200
EMBED = 64
LANES = 128
B = BATCH * HIST

_info = plsc.get_sparse_core_info()
NC, NS = _info.num_cores, _info.num_subcores
NW = NC * NS
BPW = BATCH // NW
NCH = BPW
NBUF = 2
VW = 16
assert NCH % NBUF == 0


def _body(x_hbm, table_hbm, out_hbm, idx_v, rows, rows64, *sems):
    sg, ss = sems[:NBUF], sems[NBUF:]
    wid = lax.axis_index("s") * NC + lax.axis_index("c")
    base = wid * BPW

    pltpu.sync_copy(x_hbm.at[pl.ds(base * HIST, NCH * HIST)], idx_v)

    def start_gather(j, b):
        idx = idx_v.at[pl.ds(j * HIST, HIST)]
        pltpu.async_copy(table_hbm.at[idx], rows.at[b], sg[b])

    def wait_gather(b):
        pltpu.make_async_copy(table_hbm.at[pl.ds(0, HIST)], rows.at[b], sg[b]).wait()

    def compact(b):
        @pl.loop(0, HIST, step=4)
        def _rows(r):
            for dr in range(4):
                for q in range(EMBED // VW):
                    c = q * VW
                    rows64[b, r + dr, pl.ds(c, VW)] = rows[b, r + dr, pl.ds(c, VW)]

    def start_store(j, b):
        pltpu.async_copy(rows64.at[b], out_hbm.at[base + j], ss[b])

    def wait_store(b):
        pltpu.make_async_copy(rows64.at[b], out_hbm.at[0], ss[b]).wait()

    for b in range(NBUF):
        start_gather(b, b)

    @pl.loop(0, NCH - NBUF, step=NBUF)
    def _loop(i):
        for b in range(NBUF):
            wait_gather(b)
            compact(b)
            start_store(i + b, b)
        for b in range(NBUF):
            wait_store(b)
            start_gather(i + NBUF + b, b)

    i0 = NCH - NBUF
    for b in range(NBUF):
        wait_gather(b)
        compact(b)
        start_store(i0 + b, b)
    for b in range(NBUF):
        wait_store(b)


_mesh = plsc.VectorSubcoreMesh(core_axis_name="c", subcore_axis_name="s")

_emb = functools.partial(
    pl.kernel,
    out_type=jax.ShapeDtypeStruct((BATCH, HIST, EMBED), jnp.float32),
    mesh=_mesh,
    scratch_types=[
        pltpu.VMEM((NCH * HIST,), jnp.int32),
        pltpu.VMEM((NBUF, HIST, LANES), jnp.float32),
        pltpu.VMEM((NBUF, HIST, EMBED), jnp.float32),
    ] + [pltpu.SemaphoreType.DMA] * (2 * NBUF),
    compiler_params=pltpu.CompilerParams(use_tc_tiling_on_sc=True),
)(_body)


def kernel(x, table):
    tp = jnp.pad(table, ((0, 0), (0, LANES - EMBED)))
    return _emb(x.reshape(B).astype(jnp.int32), tp)


# speedup vs baseline: 1.1955x; 1.1955x over previous
import functools

import jax
import jax.numpy as jnp
from jax import lax
from jax.experimental import pallas as pl
from jax.experimental.pallas import tpu as pltpu
from jax.experimental.pallas import tpu_sc as plsc

BATCH = 4096
HIST = 200
EMBED = 64
LANES = 128
B = BATCH * HIST

_info = plsc.get_sparse_core_info()
NC, NS = _info.num_cores, _info.num_subcores
NW = NC * NS
BPW = BATCH // NW
NCH = BPW
NBUF = 4
assert NCH % NBUF == 0
assert NCH * HIST * 4 + NBUF * HIST * LANES * 4 <= 524284


def _body(x_hbm, table_hbm, out_hbm, idx_v, rows, *sems):
    sg, ss = sems[:NBUF], sems[NBUF:]
    wid = lax.axis_index("s") * NC + lax.axis_index("c")
    base = wid * BPW

    pltpu.sync_copy(x_hbm.at[pl.ds(base * HIST, NCH * HIST)], idx_v)

    def start_gather(j, b):
        idx = idx_v.at[pl.ds(j * HIST, HIST)]
        pltpu.async_copy(table_hbm.at[idx], rows.at[b], sg[b])

    def wait_gather(b):
        pltpu.make_async_copy(table_hbm.at[pl.ds(0, HIST)], rows.at[b], sg[b]).wait()

    def start_store(j, b):
        pltpu.async_copy(rows.at[b], out_hbm.at[base + j], ss[b])

    def wait_store(b):
        pltpu.make_async_copy(rows.at[b], out_hbm.at[0], ss[b]).wait()

    for b in range(NBUF):
        start_gather(b, b)

    @pl.loop(0, NCH - NBUF, step=NBUF)
    def _loop(i):
        for b in range(NBUF):
            wait_gather(b)
            start_store(i + b, b)
        for b in range(NBUF):
            wait_store(b)
            start_gather(i + NBUF + b, b)

    i0 = NCH - NBUF
    for b in range(NBUF):
        wait_gather(b)
        start_store(i0 + b, b)
    for b in range(NBUF):
        wait_store(b)


_mesh = plsc.VectorSubcoreMesh(core_axis_name="c", subcore_axis_name="s")

_emb = functools.partial(
    pl.kernel,
    out_type=jax.ShapeDtypeStruct((BATCH, HIST, LANES), jnp.float32),
    mesh=_mesh,
    scratch_types=[
        pltpu.VMEM((NCH * HIST,), jnp.int32),
        pltpu.VMEM((NBUF, HIST, LANES), jnp.float32),
    ] + [pltpu.SemaphoreType.DMA] * (2 * NBUF),
    compiler_params=pltpu.CompilerParams(use_tc_tiling_on_sc=True),
)(_body)


def kernel(x, table):
    tp = jnp.pad(table, ((0, 0), (0, LANES - EMBED)))
    return _emb(x.reshape(B).astype(jnp.int32), tp)[:, :, :EMBED]
